# baseline (speedup 1.0000x reference)
import functools

import jax
import jax.numpy as jnp
from jax import lax
from jax.experimental import pallas as pl
from jax.experimental.pallas import tpu as pltpu
from jax.experimental.pallas import tpu_sc as plsc

MAX_ATOMS = 50
N_GRAPH_FEAT = 30
N_ATOM_FEAT = 75
N_ATOMS = 12800
LAYER_SIZE = 64
JPAD = 64
PAD_SLOT = 51
B = 256
SUBS = 2
BLK = B * SUBS
NB = N_ATOMS // BLK
PROJ_ROWS = 1600
GATHER_WINDOW = 128


def _proj_body(af_ref, w_ref, b_ref, z_ref):
    z_ref[...] = (
        lax.dot(af_ref[...], w_ref[...], preferred_element_type=jnp.float32)
        + b_ref[...]
    )


def _project_atoms(atom_features, w0a, b0):
    return pl.pallas_call(
        _proj_body,
        grid=(N_ATOMS // PROJ_ROWS,),
        in_specs=[
            pl.BlockSpec((PROJ_ROWS, N_ATOM_FEAT), lambda i: (i, 0)),
            pl.BlockSpec((N_ATOM_FEAT, LAYER_SIZE), lambda i: (0, 0)),
            pl.BlockSpec((1, LAYER_SIZE), lambda i: (0, 0)),
        ],
        out_specs=pl.BlockSpec((PROJ_ROWS, LAYER_SIZE), lambda i: (i, 0)),
        out_shape=jax.ShapeDtypeStruct((N_ATOMS, LAYER_SIZE), jnp.float32),
    )(atom_features, w0a, b0)


def _sc_gather(table, idx_flat):
    n = idx_flat.shape[0]
    idx2 = idx_flat.reshape(1, n)
    mesh = plsc.VectorSubcoreMesh(core_axis_name="core", subcore_axis_name="subcore")

    @functools.partial(
        pl.kernel,
        out_type=jax.ShapeDtypeStruct((n, LAYER_SIZE), jnp.float32),
        mesh=mesh,
        compiler_params=pltpu.CompilerParams(use_tc_tiling_on_sc=False),
    )
    def k(table_hbm, i_hbm, o_hbm):
        def body(i_vmem, o_vmem):
            pltpu.sync_copy(table_hbm.at[i_vmem.at[0]], o_vmem)

        pltpu.emit_pipeline(
            body,
            grid=(n // GATHER_WINDOW,),
            in_specs=[pl.BlockSpec((1, GATHER_WINDOW), index_map=lambda i: (0, i))],
            out_specs=[
                pl.BlockSpec((GATHER_WINDOW, LAYER_SIZE), index_map=lambda i: (i, 0))
            ],
            core_axis_name=("core", "subcore"),
            dimension_semantics=(pltpu.PARALLEL,),
        )(i_hbm, o_hbm)

    return k(table, idx2)


def _dag_body(zg_ref, par_ref, w0_ref, w1_ref, b1_ref, out_ref, g_ref):
    g_ref[...] = jnp.zeros_like(g_ref)

    def one_round(k, a0):
        par = par_ref[k, pl.ds(a0, B), :]
        tgt = par[:, 0:1]
        idx64 = jnp.pad(par[:, 1:], ((0, 0), (0, JPAD - (MAX_ATOMS - 1))),
                        constant_values=PAD_SLOT)
        idx128 = jnp.concatenate([idx64, idx64 + JPAD], axis=1)

        parts = []
        for p in range(N_GRAPH_FEAT // 2):
            gpair = g_ref[pl.ds(a0, B), p * 128:(p + 1) * 128]
            parts.append(jnp.take_along_axis(gpair, idx128, axis=-1))
        gf = jnp.concatenate(parts, axis=1)

        h = jnp.maximum(
            zg_ref[k, pl.ds(a0, B), :]
            + lax.dot(gf.astype(jnp.bfloat16), w0_ref[...],
                      preferred_element_type=jnp.float32),
            0.0,
        )
        out = jnp.maximum(
            lax.dot(h.astype(jnp.bfloat16), w1_ref[...],
                    preferred_element_type=jnp.float32) + b1_ref[...],
            0.0,
        )

        lane = lax.broadcasted_iota(jnp.int32, (B, 128), 1)
        cond = (lane & (JPAD - 1)) == tgt
        lo = lane < JPAD
        for p in range(N_GRAPH_FEAT // 2):
            gpair = g_ref[pl.ds(a0, B), p * 128:(p + 1) * 128]
            val = jnp.where(lo, out[:, 2 * p:2 * p + 1],
                            out[:, 2 * p + 1:2 * p + 2])
            g_ref[pl.ds(a0, B), p * 128:(p + 1) * 128] = jnp.where(cond, val, gpair)
        out_ref[pl.ds(a0, B), :] = out

    def loop_body(k, carry):
        for sub in range(SUBS):
            one_round(k, sub * B)
        return carry

    lax.fori_loop(0, MAX_ATOMS, loop_body, 0)


def _dag_sweep(zg, parT, w0g, w1, b1):
    return pl.pallas_call(
        _dag_body,
        grid=(NB,),
        in_specs=[
            pl.BlockSpec((MAX_ATOMS, BLK, LAYER_SIZE), lambda b: (0, b, 0)),
            pl.BlockSpec((MAX_ATOMS, BLK, MAX_ATOMS), lambda b: (0, b, 0)),
            pl.BlockSpec((N_GRAPH_FEAT * JPAD, LAYER_SIZE), lambda b: (0, 0)),
            pl.BlockSpec((LAYER_SIZE, N_GRAPH_FEAT), lambda b: (0, 0)),
            pl.BlockSpec((1, N_GRAPH_FEAT), lambda b: (0, 0)),
        ],
        out_specs=pl.BlockSpec((BLK, N_GRAPH_FEAT), lambda b: (b, 0)),
        out_shape=jax.ShapeDtypeStruct((N_ATOMS, N_GRAPH_FEAT), jnp.float32),
        scratch_shapes=[pltpu.VMEM((BLK, N_GRAPH_FEAT * JPAD), jnp.float32)],
        compiler_params=pltpu.CompilerParams(
            dimension_semantics=("parallel",),
        ),
    )(zg, parT, w0g, w1, b1)


def kernel(atom_features, parents, calculation_orders, calculation_masks,
           n_atoms, W0, b0, W1, b1):
    del calculation_masks, n_atoms
    w0a = W0[:N_ATOM_FEAT]
    w0g = W0[N_ATOM_FEAT:].reshape(MAX_ATOMS - 1, N_GRAPH_FEAT, LAYER_SIZE)
    w0g = jnp.pad(
        w0g.transpose(1, 0, 2), ((0, 0), (0, JPAD - (MAX_ATOMS - 1)), (0, 0))
    ).reshape(N_GRAPH_FEAT * JPAD, LAYER_SIZE)

    z = _project_atoms(atom_features, w0a, b0.reshape(1, LAYER_SIZE))
    idx_flat = calculation_orders.T.reshape(-1)
    zg = _sc_gather(z, idx_flat).reshape(MAX_ATOMS, N_ATOMS, LAYER_SIZE)
    parT = parents.transpose(1, 0, 2)
    return _dag_sweep(zg, parT, w0g.astype(jnp.bfloat16),
                      W1.astype(jnp.bfloat16), b1.reshape(1, N_GRAPH_FEAT))


# speedup vs baseline: 1.6813x; 1.6813x over previous
import functools

import jax
import jax.numpy as jnp
from jax import lax
from jax.experimental import pallas as pl
from jax.experimental.pallas import tpu as pltpu
from jax.experimental.pallas import tpu_sc as plsc

MAX_ATOMS = 50
N_GRAPH_FEAT = 30
N_ATOM_FEAT = 75
N_ATOMS = 12800
LAYER_SIZE = 64
JPAD = 64
PAD_SLOT = 51
B = 512
NB = N_ATOMS // B
PROJ_ROWS = 1600
GATHER_WINDOW = 128


def _proj_body(af_ref, w_ref, b_ref, z_ref):
    z_ref[...] = (
        lax.dot(af_ref[...], w_ref[...], preferred_element_type=jnp.float32)
        + b_ref[...]
    )


def _project_atoms(atom_features, w0a, b0):
    return pl.pallas_call(
        _proj_body,
        grid=(N_ATOMS // PROJ_ROWS,),
        in_specs=[
            pl.BlockSpec((PROJ_ROWS, N_ATOM_FEAT), lambda i: (i, 0)),
            pl.BlockSpec((N_ATOM_FEAT, LAYER_SIZE), lambda i: (0, 0)),
            pl.BlockSpec((1, LAYER_SIZE), lambda i: (0, 0)),
        ],
        out_specs=pl.BlockSpec((PROJ_ROWS, LAYER_SIZE), lambda i: (i, 0)),
        out_shape=jax.ShapeDtypeStruct((N_ATOMS, LAYER_SIZE), jnp.float32),
    )(atom_features, w0a, b0)


def _sc_gather(table, idx_flat):
    n = idx_flat.shape[0]
    idx2 = idx_flat.reshape(1, n)
    mesh = plsc.VectorSubcoreMesh(core_axis_name="core", subcore_axis_name="subcore")

    @functools.partial(
        pl.kernel,
        out_type=jax.ShapeDtypeStruct((n, LAYER_SIZE), jnp.float32),
        mesh=mesh,
        compiler_params=pltpu.CompilerParams(use_tc_tiling_on_sc=False),
    )
    def k(table_hbm, i_hbm, o_hbm):
        def body(i_vmem, o_vmem):
            pltpu.sync_copy(table_hbm.at[i_vmem.at[0]], o_vmem)

        pltpu.emit_pipeline(
            body,
            grid=(n // GATHER_WINDOW,),
            in_specs=[pl.BlockSpec((1, GATHER_WINDOW), index_map=lambda i: (0, i))],
            out_specs=[
                pl.BlockSpec((GATHER_WINDOW, LAYER_SIZE), index_map=lambda i: (i, 0))
            ],
            core_axis_name=("core", "subcore"),
            dimension_semantics=(pltpu.PARALLEL,),
        )(i_hbm, o_hbm)

    return k(table, idx2)


def _dag_body(zg_ref, par_ref, w0_ref, w1_ref, b1_ref, out_ref, g_ref):
    k = pl.program_id(1)

    @pl.when(k == 0)
    def _():
        g_ref[...] = jnp.zeros_like(g_ref)

    par = par_ref[0]
    tgt = par[:, 0:1]
    idx64 = jnp.pad(par[:, 1:], ((0, 0), (0, JPAD - (MAX_ATOMS - 1))),
                    constant_values=PAD_SLOT)
    idx128 = jnp.concatenate([idx64, idx64 + JPAD], axis=1)

    parts = []
    for p in range(N_GRAPH_FEAT // 2):
        gpair = g_ref[:, p * 128:(p + 1) * 128]
        parts.append(jnp.take_along_axis(gpair, idx128, axis=-1))
    gf = jnp.concatenate(parts, axis=1)

    h = jnp.maximum(
        zg_ref[0]
        + lax.dot(gf.astype(jnp.bfloat16), w0_ref[...],
                  preferred_element_type=jnp.float32),
        0.0,
    )
    out = jnp.maximum(
        lax.dot(h.astype(jnp.bfloat16), w1_ref[...],
                preferred_element_type=jnp.float32) + b1_ref[...],
        0.0,
    )

    lane = lax.broadcasted_iota(jnp.int32, (B, 128), 1)
    cond = (lane & (JPAD - 1)) == tgt
    ge64 = (lane >= JPAD).astype(jnp.int32)
    outp = jnp.pad(out, ((0, 0), (0, 128 - N_GRAPH_FEAT)))
    for p in range(N_GRAPH_FEAT // 2):
        gpair = g_ref[:, p * 128:(p + 1) * 128]
        val = jnp.take_along_axis(outp, ge64 + 2 * p, axis=-1)
        g_ref[:, p * 128:(p + 1) * 128] = jnp.where(cond, val, gpair)
    out_ref[...] = out


def _dag_sweep(zg, parT, w0g, w1, b1):
    return pl.pallas_call(
        _dag_body,
        grid=(NB, MAX_ATOMS),
        in_specs=[
            pl.BlockSpec((1, B, LAYER_SIZE), lambda b, k: (k, b, 0)),
            pl.BlockSpec((1, B, MAX_ATOMS), lambda b, k: (k, b, 0)),
            pl.BlockSpec((N_GRAPH_FEAT * JPAD, LAYER_SIZE), lambda b, k: (0, 0)),
            pl.BlockSpec((LAYER_SIZE, N_GRAPH_FEAT), lambda b, k: (0, 0)),
            pl.BlockSpec((1, N_GRAPH_FEAT), lambda b, k: (0, 0)),
        ],
        out_specs=pl.BlockSpec((B, N_GRAPH_FEAT), lambda b, k: (b, 0)),
        out_shape=jax.ShapeDtypeStruct((N_ATOMS, N_GRAPH_FEAT), jnp.float32),
        scratch_shapes=[pltpu.VMEM((B, N_GRAPH_FEAT * JPAD), jnp.float32)],
        compiler_params=pltpu.CompilerParams(
            dimension_semantics=("parallel", "arbitrary"),
        ),
    )(zg, parT, w0g, w1, b1)


def kernel(atom_features, parents, calculation_orders, calculation_masks,
           n_atoms, W0, b0, W1, b1):
    del calculation_masks, n_atoms
    w0a = W0[:N_ATOM_FEAT]
    w0g = W0[N_ATOM_FEAT:].reshape(MAX_ATOMS - 1, N_GRAPH_FEAT, LAYER_SIZE)
    w0g = jnp.pad(
        w0g.transpose(1, 0, 2), ((0, 0), (0, JPAD - (MAX_ATOMS - 1)), (0, 0))
    ).reshape(N_GRAPH_FEAT * JPAD, LAYER_SIZE)

    z = _project_atoms(atom_features, w0a, b0.reshape(1, LAYER_SIZE))
    idx_flat = calculation_orders.T.reshape(-1)
    zg = _sc_gather(z, idx_flat).reshape(MAX_ATOMS, N_ATOMS, LAYER_SIZE)
    parT = parents.transpose(1, 0, 2)
    return _dag_sweep(zg, parT, w0g.astype(jnp.bfloat16),
                      W1.astype(jnp.bfloat16), b1.reshape(1, N_GRAPH_FEAT))
